# Initial kernel scaffold; baseline (speedup 1.0000x reference)
import jax
import jax.numpy as jnp
from jax.experimental import pallas as pl


def kernel(x, edge_index, k, W_weight, W_bias, prelu_a):
    raise NotImplementedError("write your pallas kernel here")


# speedup vs baseline: 3.7156x; 3.7156x over previous
import functools

import jax
import jax.numpy as jnp
from jax import lax
from jax.experimental import pallas as pl
from jax.experimental.pallas import tpu as pltpu
from jax.experimental.pallas import tpu_sc as plsc

N_NODES = 10000
N_PAD = 10240
D_FEAT = 256
N_EDGES = 160000
HALF = 128
NC = 2
NS = 16
ROWS_PER_TILE = N_PAD // NS

HOP_CH = 64
HOP_EPT = 10112
HOP_NCH = HOP_EPT // HOP_CH
DEG_CH = 64
DEG_EPT = 5120
DEG_NCH = DEG_EPT // DEG_CH
STG = 64


def _make_deg_kernel():
    mesh = plsc.VectorSubcoreMesh(core_axis_name="c", subcore_axis_name="s")

    @functools.partial(
        pl.kernel,
        mesh=mesh,
        out_type=jax.ShapeDtypeStruct((NC, N_PAD, HALF), jnp.float32),
        scratch_types=[
            pltpu.VMEM((DEG_NCH, DEG_CH), jnp.int32),
            pltpu.VMEM((DEG_CH, HALF), jnp.float32),
            pltpu.VMEM((STG, HALF), jnp.float32),
            pltpu.VMEM_SHARED((N_PAD, HALF), jnp.float32),
        ],
    )
    def deg_k(dst_hbm, ones_hbm, zeros_hbm, out_hbm,
              dst_v, ones_v, stage_v, acc_sh):
        c = lax.axis_index("c")
        s = lax.axis_index("s")
        wid = c * NS + s
        base = s * ROWS_PER_TILE

        pltpu.sync_copy(ones_hbm, ones_v)
        pltpu.sync_copy(zeros_hbm, stage_v)

        def zs(i, _):
            pltpu.sync_copy(stage_v, acc_sh.at[pl.ds(base + i * STG, STG)])
            return 0
        lax.fori_loop(0, ROWS_PER_TILE // STG, zs, 0)
        pltpu.sync_copy(dst_hbm.at[wid], dst_v)
        plsc.subcore_barrier()

        def step(j, _):
            pltpu.sync_copy(ones_v, acc_sh.at[dst_v.at[j]], add=True)
            return 0
        lax.fori_loop(0, DEG_NCH, step, 0)
        plsc.subcore_barrier()

        def outp(i, _):
            pltpu.sync_copy(acc_sh.at[pl.ds(base + i * STG, STG)], stage_v)
            pltpu.sync_copy(stage_v, out_hbm.at[c, pl.ds(base + i * STG, STG)])
            return 0
        lax.fori_loop(0, ROWS_PER_TILE // STG, outp, 0)

    return deg_k


def _make_hop_kernel():
    mesh = plsc.VectorSubcoreMesh(core_axis_name="c", subcore_axis_name="s")

    @functools.partial(
        pl.kernel,
        mesh=mesh,
        out_type=[
            jax.ShapeDtypeStruct((N_PAD, HALF), jnp.float32),
            jax.ShapeDtypeStruct((N_PAD, HALF), jnp.float32),
        ],
        scratch_types=[
            pltpu.VMEM((HOP_NCH, HOP_CH), jnp.int32),
            pltpu.VMEM((HOP_NCH, HOP_CH), jnp.int32),
            pltpu.VMEM((HOP_CH, HALF), jnp.float32),
            pltpu.VMEM_SHARED((N_PAD, HALF), jnp.float32),
            pltpu.SemaphoreType.DMA,
        ],
    )
    def hop_k(h0_hbm, h1_hbm, src_hbm, dst_hbm, zeros_hbm, o0_hbm, o1_hbm,
              src_v, dst_v, rows_v, acc_sh, sem):
        c = lax.axis_index("c")
        s = lax.axis_index("s")
        base = s * ROWS_PER_TILE

        pltpu.sync_copy(zeros_hbm, rows_v)

        def zs(i, _):
            pltpu.sync_copy(rows_v, acc_sh.at[pl.ds(base + i * STG, STG)])
            return 0
        lax.fori_loop(0, ROWS_PER_TILE // STG, zs, 0)
        pltpu.sync_copy(src_hbm.at[s], src_v)
        pltpu.sync_copy(dst_hbm.at[s], dst_v)
        plsc.subcore_barrier()

        def run(h_hbm):
            def step(j, _):
                pltpu.async_copy(h_hbm.at[src_v.at[j]], rows_v, sem).wait()
                pltpu.sync_copy(rows_v, acc_sh.at[dst_v.at[j]], add=True)
                return 0
            lax.fori_loop(0, HOP_NCH, step, 0)

        @pl.when(c == 0)
        def _():
            run(h0_hbm)

        @pl.when(c == 1)
        def _():
            run(h1_hbm)

        plsc.subcore_barrier()

        def outp(o_hbm):
            def ostep(i, _):
                pltpu.sync_copy(acc_sh.at[pl.ds(base + i * STG, STG)], rows_v)
                pltpu.sync_copy(rows_v, o_hbm.at[pl.ds(base + i * STG, STG)])
                return 0
            lax.fori_loop(0, ROWS_PER_TILE // STG, ostep, 0)

        @pl.when(c == 0)
        def _():
            outp(o0_hbm)

        @pl.when(c == 1)
        def _():
            outp(o1_hbm)

    return hop_k


def _tc_body(s0_ref, s1_ref, inv_ref, w_ref, b_ref, a_ref, o0_ref, o1_ref):
    agg = jnp.concatenate([s0_ref[...], s1_ref[...]], axis=1) * inv_ref[...]
    lin = lax.dot_general(agg, w_ref[...], (((1,), (1,)), ((), ())),
                          preferred_element_type=jnp.float32)
    lin = lin + b_ref[...]
    a = a_ref[0, 0]
    out = jnp.maximum(lin, 0.0) + a * jnp.minimum(lin, 0.0)
    o0_ref[...] = out[:, :HALF]
    o1_ref[...] = out[:, HALF:]


def _tc_post(s0, s1, inv, w, b2, a2):
    r = 1024
    return pl.pallas_call(
        _tc_body,
        grid=(N_PAD // r,),
        in_specs=[
            pl.BlockSpec((r, HALF), lambda i: (i, 0)),
            pl.BlockSpec((r, HALF), lambda i: (i, 0)),
            pl.BlockSpec((r, 1), lambda i: (i, 0)),
            pl.BlockSpec((D_FEAT, D_FEAT), lambda i: (0, 0)),
            pl.BlockSpec((1, D_FEAT), lambda i: (0, 0)),
            pl.BlockSpec((1, 1), lambda i: (0, 0)),
        ],
        out_specs=[
            pl.BlockSpec((r, HALF), lambda i: (i, 0)),
            pl.BlockSpec((r, HALF), lambda i: (i, 0)),
        ],
        out_shape=[jax.ShapeDtypeStruct((N_PAD, HALF), jnp.float32)] * 2,
    )(s0, s1, inv, w, b2, a2)


_deg_kernel = _make_deg_kernel()
_hop_kernel = _make_hop_kernel()


def _pad_edges(idx, per_tile, padded_per_tile, n_tiles, nch, ch):
    t = idx.reshape(n_tiles, per_tile)
    t = jnp.pad(t, ((0, 0), (0, padded_per_tile - per_tile)),
                constant_values=N_NODES)
    return t.reshape(n_tiles, nch, ch)


def kernel(x, edge_index, k, W_weight, W_bias, prelu_a):
    ei = edge_index.astype(jnp.int32)
    src = _pad_edges(ei[0], N_EDGES // NS, HOP_EPT, NS, HOP_NCH, HOP_CH)
    dst = _pad_edges(ei[1], N_EDGES // NS, HOP_EPT, NS, HOP_NCH, HOP_CH)
    dst2 = _pad_edges(ei[1], N_EDGES // (NC * NS), DEG_EPT, NC * NS,
                      DEG_NCH, DEG_CH)
    h0 = jnp.pad(x[:, :HALF], ((0, N_PAD - N_NODES), (0, 0)))
    h1 = jnp.pad(x[:, HALF:], ((0, N_PAD - N_NODES), (0, 0)))

    ones_c = jnp.ones((DEG_CH, HALF), jnp.float32)
    zeros_c = jnp.zeros((STG, HALF), jnp.float32)
    degp = _deg_kernel(dst2, ones_c, zeros_c)
    deg = jnp.maximum(degp[0, :, 0] + degp[1, :, 0], 1.0)
    inv = (1.0 / deg).reshape(N_PAD, 1)
    b2 = W_bias.reshape(1, D_FEAT)
    a2 = prelu_a.reshape(1, 1)

    def hop(_, carry):
        c0, c1 = carry
        s0, s1 = _hop_kernel(c0, c1, src, dst, zeros_c)
        return tuple(_tc_post(s0, s1, inv, W_weight, b2, a2))

    h0, h1 = lax.fori_loop(0, k, hop, (h0, h1))
    return jnp.concatenate([h0[:N_NODES], h1[:N_NODES]], axis=1)


# speedup vs baseline: 4.4544x; 1.1988x over previous
import functools

import jax
import jax.numpy as jnp
from jax import lax
from jax.experimental import pallas as pl
from jax.experimental.pallas import tpu as pltpu
from jax.experimental.pallas import tpu_sc as plsc

N_NODES = 10000
N_PAD = 10240
D_FEAT = 256
N_EDGES = 160000
HALF = 128
NC = 2
NS = 16
ROWS_PER_TILE = N_PAD // NS

HOP_CH = 80
HOP_EPT = 10080
HOP_NCH = HOP_EPT // HOP_CH
HOP_NT = HOP_NCH // 2
DEG_CH = 64
DEG_EPT = 5120
DEG_NCH = DEG_EPT // DEG_CH
STG = 64
HSTG = 80


def _make_deg_kernel():
    mesh = plsc.VectorSubcoreMesh(core_axis_name="c", subcore_axis_name="s")

    @functools.partial(
        pl.kernel,
        mesh=mesh,
        out_type=jax.ShapeDtypeStruct((NC, N_PAD, HALF), jnp.float32),
        scratch_types=[
            pltpu.VMEM((DEG_NCH, DEG_CH), jnp.int32),
            pltpu.VMEM((DEG_CH, HALF), jnp.float32),
            pltpu.VMEM((STG, HALF), jnp.float32),
            pltpu.VMEM_SHARED((N_PAD, HALF), jnp.float32),
        ],
    )
    def deg_k(dst_hbm, ones_hbm, zeros_hbm, out_hbm,
              dst_v, ones_v, stage_v, acc_sh):
        c = lax.axis_index("c")
        s = lax.axis_index("s")
        wid = c * NS + s
        base = s * ROWS_PER_TILE

        pltpu.sync_copy(ones_hbm, ones_v)
        pltpu.sync_copy(zeros_hbm, stage_v)

        def zs(i, _):
            pltpu.sync_copy(stage_v, acc_sh.at[pl.ds(base + i * STG, STG)])
            return 0
        lax.fori_loop(0, ROWS_PER_TILE // STG, zs, 0)
        pltpu.sync_copy(dst_hbm.at[wid], dst_v)
        plsc.subcore_barrier()

        def step(j, _):
            pltpu.sync_copy(ones_v, acc_sh.at[dst_v.at[j]], add=True)
            return 0
        lax.fori_loop(0, DEG_NCH, step, 0)
        plsc.subcore_barrier()

        def outp(i, _):
            pltpu.sync_copy(acc_sh.at[pl.ds(base + i * STG, STG)], stage_v)
            pltpu.sync_copy(stage_v, out_hbm.at[c, pl.ds(base + i * STG, STG)])
            return 0
        lax.fori_loop(0, ROWS_PER_TILE // STG, outp, 0)

    return deg_k


def _make_hop_kernel():
    mesh = plsc.VectorSubcoreMesh(core_axis_name="c", subcore_axis_name="s")

    @functools.partial(
        pl.kernel,
        mesh=mesh,
        out_type=[
            jax.ShapeDtypeStruct((N_PAD, HALF), jnp.float32),
            jax.ShapeDtypeStruct((N_PAD, HALF), jnp.float32),
        ],
        scratch_types=[
            pltpu.VMEM((HOP_CH,), jnp.int32),
            pltpu.VMEM((HOP_CH,), jnp.int32),
            pltpu.VMEM((HOP_CH,), jnp.int32),
            pltpu.VMEM((HOP_CH,), jnp.int32),
            pltpu.VMEM((HOP_CH, HALF), jnp.float32),
            pltpu.VMEM((HOP_CH, HALF), jnp.float32),
            pltpu.VMEM_SHARED((N_PAD, HALF), jnp.float32),
            pltpu.SemaphoreType.DMA,
            pltpu.SemaphoreType.DMA,
        ],
    )
    def hop_k(h0_hbm, h1_hbm, src_hbm, dst_hbm, zeros_hbm, o0_hbm, o1_hbm,
              isa, isb, ida, idb, rows_a, rows_b, acc_sh, sem_a, sem_b):
        c = lax.axis_index("c")
        s = lax.axis_index("s")
        base = s * ROWS_PER_TILE

        pltpu.sync_copy(zeros_hbm, rows_a)

        def zs(i, _):
            pltpu.sync_copy(rows_a, acc_sh.at[pl.ds(base + i * HSTG, HSTG)])
            return 0
        lax.fori_loop(0, ROWS_PER_TILE // HSTG, zs, 0)
        plsc.subcore_barrier()

        def run(h_hbm):
            def wait_a():
                pltpu.make_async_copy(
                    h_hbm.at[pl.ds(0, HOP_CH)], rows_a, sem_a).wait()

            def wait_b():
                pltpu.make_async_copy(
                    h_hbm.at[pl.ds(0, HOP_CH)], rows_b, sem_b).wait()

            pltpu.sync_copy(src_hbm.at[s, 0], isa)
            pltpu.sync_copy(dst_hbm.at[s, 0], ida)
            pltpu.async_copy(h_hbm.at[isa], rows_a, sem_a)
            pltpu.sync_copy(src_hbm.at[s, 1], isb)
            pltpu.sync_copy(dst_hbm.at[s, 1], idb)

            def step2(t, _):
                ja = 2 * t
                jb = ja + 1
                wait_a()
                pltpu.async_copy(h_hbm.at[isb], rows_b, sem_b)
                pltpu.sync_copy(rows_a, acc_sh.at[ida], add=True)

                @pl.when(t + 1 < HOP_NT)
                def _():
                    pltpu.sync_copy(src_hbm.at[s, ja + 2], isa)
                    pltpu.sync_copy(dst_hbm.at[s, ja + 2], ida)

                wait_b()

                @pl.when(t + 1 < HOP_NT)
                def _():
                    pltpu.async_copy(h_hbm.at[isa], rows_a, sem_a)

                pltpu.sync_copy(rows_b, acc_sh.at[idb], add=True)

                @pl.when(t + 1 < HOP_NT)
                def _():
                    pltpu.sync_copy(src_hbm.at[s, jb + 2], isb)
                    pltpu.sync_copy(dst_hbm.at[s, jb + 2], idb)

                return 0
            lax.fori_loop(0, HOP_NT, step2, 0)

        @pl.when(c == 0)
        def _():
            run(h0_hbm)

        @pl.when(c == 1)
        def _():
            run(h1_hbm)

        plsc.subcore_barrier()

        def outp(o_hbm):
            def ostep(i, _):
                pltpu.sync_copy(acc_sh.at[pl.ds(base + i * HSTG, HSTG)], rows_a)
                pltpu.sync_copy(rows_a, o_hbm.at[pl.ds(base + i * HSTG, HSTG)])
                return 0
            lax.fori_loop(0, ROWS_PER_TILE // HSTG, ostep, 0)

        @pl.when(c == 0)
        def _():
            outp(o0_hbm)

        @pl.when(c == 1)
        def _():
            outp(o1_hbm)

    return hop_k


def _tc_body(s0_ref, s1_ref, inv_ref, w_ref, b_ref, a_ref, o0_ref, o1_ref):
    agg = jnp.concatenate([s0_ref[...], s1_ref[...]], axis=1) * inv_ref[...]
    lin = lax.dot_general(agg, w_ref[...], (((1,), (1,)), ((), ())),
                          preferred_element_type=jnp.float32)
    lin = lin + b_ref[...]
    a = a_ref[0, 0]
    out = jnp.maximum(lin, 0.0) + a * jnp.minimum(lin, 0.0)
    o0_ref[...] = out[:, :HALF]
    o1_ref[...] = out[:, HALF:]


def _tc_post(s0, s1, inv, w, b2, a2):
    r = 1024
    return pl.pallas_call(
        _tc_body,
        grid=(N_PAD // r,),
        in_specs=[
            pl.BlockSpec((r, HALF), lambda i: (i, 0)),
            pl.BlockSpec((r, HALF), lambda i: (i, 0)),
            pl.BlockSpec((r, 1), lambda i: (i, 0)),
            pl.BlockSpec((D_FEAT, D_FEAT), lambda i: (0, 0)),
            pl.BlockSpec((1, D_FEAT), lambda i: (0, 0)),
            pl.BlockSpec((1, 1), lambda i: (0, 0)),
        ],
        out_specs=[
            pl.BlockSpec((r, HALF), lambda i: (i, 0)),
            pl.BlockSpec((r, HALF), lambda i: (i, 0)),
        ],
        out_shape=[jax.ShapeDtypeStruct((N_PAD, HALF), jnp.float32)] * 2,
    )(s0, s1, inv, w, b2, a2)


_deg_kernel = _make_deg_kernel()
_hop_kernel = _make_hop_kernel()


def _pad_edges(idx, per_tile, padded_per_tile, n_tiles, nch, ch):
    t = idx.reshape(n_tiles, per_tile)
    t = jnp.pad(t, ((0, 0), (0, padded_per_tile - per_tile)),
                constant_values=N_NODES)
    return t.reshape(n_tiles, nch, ch)


def kernel(x, edge_index, k, W_weight, W_bias, prelu_a):
    ei = edge_index.astype(jnp.int32)
    src = _pad_edges(ei[0], N_EDGES // NS, HOP_EPT, NS, HOP_NCH, HOP_CH)
    dst = _pad_edges(ei[1], N_EDGES // NS, HOP_EPT, NS, HOP_NCH, HOP_CH)
    dst2 = _pad_edges(ei[1], N_EDGES // (NC * NS), DEG_EPT, NC * NS,
                      DEG_NCH, DEG_CH)
    h0 = jnp.pad(x[:, :HALF], ((0, N_PAD - N_NODES), (0, 0)))
    h1 = jnp.pad(x[:, HALF:], ((0, N_PAD - N_NODES), (0, 0)))

    ones_c = jnp.ones((DEG_CH, HALF), jnp.float32)
    zeros_c = jnp.zeros((STG, HALF), jnp.float32)
    zeros_h = jnp.zeros((HSTG, HALF), jnp.float32)
    degp = _deg_kernel(dst2, ones_c, zeros_c)
    deg = jnp.maximum(degp[0, :, 0] + degp[1, :, 0], 1.0)
    inv = (1.0 / deg).reshape(N_PAD, 1)
    b2 = W_bias.reshape(1, D_FEAT)
    a2 = prelu_a.reshape(1, 1)

    def hop(_, carry):
        c0, c1 = carry
        s0, s1 = _hop_kernel(c0, c1, src, dst, zeros_h)
        return tuple(_tc_post(s0, s1, inv, W_weight, b2, a2))

    h0, h1 = lax.fori_loop(0, k, hop, (h0, h1))
    return jnp.concatenate([h0[:N_NODES], h1[:N_NODES]], axis=1)


# speedup vs baseline: 5.0142x; 1.1257x over previous
import functools

import jax
import jax.numpy as jnp
from jax import lax
from jax.experimental import pallas as pl
from jax.experimental.pallas import tpu as pltpu
from jax.experimental.pallas import tpu_sc as plsc

N_NODES = 10000
N_PAD = 10240
D_FEAT = 256
N_EDGES = 160000
HALF = 128
NC = 2
NS = 16
ROWS_PER_TILE = N_PAD // NS

HOP_CH = 80
HOP_EPT = 10080
HOP_NCH = HOP_EPT // HOP_CH
HOP_NT = HOP_NCH // 2
DEG_CH = 64
DEG_EPT = 5120
DEG_NCH = DEG_EPT // DEG_CH
STG = 64
HSTG = 80


def _make_deg_kernel():
    mesh = plsc.VectorSubcoreMesh(core_axis_name="c", subcore_axis_name="s")

    @functools.partial(
        pl.kernel,
        mesh=mesh,
        out_type=jax.ShapeDtypeStruct((NC, N_PAD, HALF), jnp.float32),
        scratch_types=[
            pltpu.VMEM((DEG_NCH, DEG_CH), jnp.int32),
            pltpu.VMEM((DEG_CH, HALF), jnp.float32),
            pltpu.VMEM((STG, HALF), jnp.float32),
            pltpu.VMEM_SHARED((N_PAD, HALF), jnp.float32),
        ],
    )
    def deg_k(dst_hbm, ones_hbm, zeros_hbm, out_hbm,
              dst_v, ones_v, stage_v, acc_sh):
        c = lax.axis_index("c")
        s = lax.axis_index("s")
        wid = c * NS + s
        base = s * ROWS_PER_TILE

        pltpu.sync_copy(ones_hbm, ones_v)
        pltpu.sync_copy(zeros_hbm, stage_v)

        def zs(i, _):
            pltpu.sync_copy(stage_v, acc_sh.at[pl.ds(base + i * STG, STG)])
            return 0
        lax.fori_loop(0, ROWS_PER_TILE // STG, zs, 0)
        pltpu.sync_copy(dst_hbm.at[wid], dst_v)
        plsc.subcore_barrier()

        def step(j, _):
            pltpu.sync_copy(ones_v, acc_sh.at[dst_v.at[j]], add=True)
            return 0
        lax.fori_loop(0, DEG_NCH, step, 0)
        plsc.subcore_barrier()

        def outp(i, _):
            pltpu.sync_copy(acc_sh.at[pl.ds(base + i * STG, STG)], stage_v)
            pltpu.sync_copy(stage_v, out_hbm.at[c, pl.ds(base + i * STG, STG)])
            return 0
        lax.fori_loop(0, ROWS_PER_TILE // STG, outp, 0)

    return deg_k


def _make_hop_kernel():
    mesh = plsc.VectorSubcoreMesh(core_axis_name="c", subcore_axis_name="s")

    @functools.partial(
        pl.kernel,
        mesh=mesh,
        out_type=[
            jax.ShapeDtypeStruct((N_PAD, HALF), jnp.float32),
            jax.ShapeDtypeStruct((N_PAD, HALF), jnp.float32),
        ],
        scratch_types=[
            pltpu.VMEM((HOP_CH,), jnp.int32),
            pltpu.VMEM((HOP_CH,), jnp.int32),
            pltpu.VMEM((HOP_CH,), jnp.int32),
            pltpu.VMEM((HOP_CH,), jnp.int32),
            pltpu.VMEM((HOP_CH, HALF), jnp.float32),
            pltpu.VMEM((HOP_CH, HALF), jnp.float32),
            pltpu.VMEM_SHARED((N_PAD, HALF), jnp.float32),
            pltpu.SemaphoreType.DMA,
            pltpu.SemaphoreType.DMA,
            pltpu.SemaphoreType.DMA,
            pltpu.SemaphoreType.DMA,
            pltpu.SemaphoreType.DMA,
            pltpu.SemaphoreType.DMA,
        ],
    )
    def hop_k(h0_hbm, h1_hbm, src_hbm, dst_hbm, zeros_hbm, o0_hbm, o1_hbm,
              isa, isb, ida, idb, rows_a, rows_b, acc_sh, sem_a, sem_b,
              sem_isa, sem_isb, sem_ida, sem_idb):
        c = lax.axis_index("c")
        s = lax.axis_index("s")
        base = s * ROWS_PER_TILE

        pltpu.sync_copy(zeros_hbm, rows_a)

        def zs(i, _):
            pltpu.sync_copy(rows_a, acc_sh.at[pl.ds(base + i * HSTG, HSTG)])
            return 0
        lax.fori_loop(0, ROWS_PER_TILE // HSTG, zs, 0)
        plsc.subcore_barrier()

        def run(h_hbm):
            def wait_a():
                pltpu.make_async_copy(
                    h_hbm.at[pl.ds(0, HOP_CH)], rows_a, sem_a).wait()

            def wait_b():
                pltpu.make_async_copy(
                    h_hbm.at[pl.ds(0, HOP_CH)], rows_b, sem_b).wait()

            def wait_idx(buf, sem):
                pltpu.make_async_copy(src_hbm.at[s, 0], buf, sem).wait()

            pltpu.async_copy(src_hbm.at[s, 0], isa, sem_isa)
            pltpu.async_copy(dst_hbm.at[s, 0], ida, sem_ida)
            pltpu.async_copy(src_hbm.at[s, 1], isb, sem_isb)
            pltpu.async_copy(dst_hbm.at[s, 1], idb, sem_idb)
            wait_idx(isa, sem_isa)
            pltpu.async_copy(h_hbm.at[isa], rows_a, sem_a)

            def step2(t, _):
                ja = 2 * t
                jb = ja + 1
                wait_a()

                @pl.when(t + 1 < HOP_NT)
                def _():
                    pltpu.async_copy(src_hbm.at[s, ja + 2], isa, sem_isa)

                wait_idx(isb, sem_isb)
                pltpu.async_copy(h_hbm.at[isb], rows_b, sem_b)
                wait_idx(ida, sem_ida)
                pltpu.sync_copy(rows_a, acc_sh.at[ida], add=True)

                @pl.when(t + 1 < HOP_NT)
                def _():
                    pltpu.async_copy(dst_hbm.at[s, ja + 2], ida, sem_ida)

                wait_b()

                @pl.when(t + 1 < HOP_NT)
                def _():
                    wait_idx(isa, sem_isa)
                    pltpu.async_copy(h_hbm.at[isa], rows_a, sem_a)
                    pltpu.async_copy(src_hbm.at[s, jb + 2], isb, sem_isb)

                wait_idx(idb, sem_idb)
                pltpu.sync_copy(rows_b, acc_sh.at[idb], add=True)

                @pl.when(t + 1 < HOP_NT)
                def _():
                    pltpu.async_copy(dst_hbm.at[s, jb + 2], idb, sem_idb)

                return 0
            lax.fori_loop(0, HOP_NT, step2, 0)

        @pl.when(c == 0)
        def _():
            run(h0_hbm)

        @pl.when(c == 1)
        def _():
            run(h1_hbm)

        plsc.subcore_barrier()

        def outp(o_hbm):
            def ostep(i, _):
                pltpu.sync_copy(acc_sh.at[pl.ds(base + i * HSTG, HSTG)], rows_a)
                pltpu.sync_copy(rows_a, o_hbm.at[pl.ds(base + i * HSTG, HSTG)])
                return 0
            lax.fori_loop(0, ROWS_PER_TILE // HSTG, ostep, 0)

        @pl.when(c == 0)
        def _():
            outp(o0_hbm)

        @pl.when(c == 1)
        def _():
            outp(o1_hbm)

    return hop_k


def _tc_body(s0_ref, s1_ref, inv_ref, w_ref, b_ref, a_ref, o0_ref, o1_ref):
    agg = jnp.concatenate([s0_ref[...], s1_ref[...]], axis=1) * inv_ref[...]
    lin = lax.dot_general(agg, w_ref[...], (((1,), (1,)), ((), ())),
                          preferred_element_type=jnp.float32)
    lin = lin + b_ref[...]
    a = a_ref[0, 0]
    out = jnp.maximum(lin, 0.0) + a * jnp.minimum(lin, 0.0)
    o0_ref[...] = out[:, :HALF]
    o1_ref[...] = out[:, HALF:]


def _tc_post(s0, s1, inv, w, b2, a2):
    r = 1024
    return pl.pallas_call(
        _tc_body,
        grid=(N_PAD // r,),
        in_specs=[
            pl.BlockSpec((r, HALF), lambda i: (i, 0)),
            pl.BlockSpec((r, HALF), lambda i: (i, 0)),
            pl.BlockSpec((r, 1), lambda i: (i, 0)),
            pl.BlockSpec((D_FEAT, D_FEAT), lambda i: (0, 0)),
            pl.BlockSpec((1, D_FEAT), lambda i: (0, 0)),
            pl.BlockSpec((1, 1), lambda i: (0, 0)),
        ],
        out_specs=[
            pl.BlockSpec((r, HALF), lambda i: (i, 0)),
            pl.BlockSpec((r, HALF), lambda i: (i, 0)),
        ],
        out_shape=[jax.ShapeDtypeStruct((N_PAD, HALF), jnp.float32)] * 2,
    )(s0, s1, inv, w, b2, a2)


_deg_kernel = _make_deg_kernel()
_hop_kernel = _make_hop_kernel()


def _pad_edges(idx, per_tile, padded_per_tile, n_tiles, nch, ch):
    t = idx.reshape(n_tiles, per_tile)
    t = jnp.pad(t, ((0, 0), (0, padded_per_tile - per_tile)),
                constant_values=N_NODES)
    return t.reshape(n_tiles, nch, ch)


def kernel(x, edge_index, k, W_weight, W_bias, prelu_a):
    ei = edge_index.astype(jnp.int32)
    src = _pad_edges(ei[0], N_EDGES // NS, HOP_EPT, NS, HOP_NCH, HOP_CH)
    dst = _pad_edges(ei[1], N_EDGES // NS, HOP_EPT, NS, HOP_NCH, HOP_CH)
    dst2 = _pad_edges(ei[1], N_EDGES // (NC * NS), DEG_EPT, NC * NS,
                      DEG_NCH, DEG_CH)
    h0 = jnp.pad(x[:, :HALF], ((0, N_PAD - N_NODES), (0, 0)))
    h1 = jnp.pad(x[:, HALF:], ((0, N_PAD - N_NODES), (0, 0)))

    ones_c = jnp.ones((DEG_CH, HALF), jnp.float32)
    zeros_c = jnp.zeros((STG, HALF), jnp.float32)
    zeros_h = jnp.zeros((HSTG, HALF), jnp.float32)
    degp = _deg_kernel(dst2, ones_c, zeros_c)
    deg = jnp.maximum(degp[0, :, 0] + degp[1, :, 0], 1.0)
    inv = (1.0 / deg).reshape(N_PAD, 1)
    b2 = W_bias.reshape(1, D_FEAT)
    a2 = prelu_a.reshape(1, 1)

    def hop(_, carry):
        c0, c1 = carry
        s0, s1 = _hop_kernel(c0, c1, src, dst, zeros_h)
        return tuple(_tc_post(s0, s1, inv, W_weight, b2, a2))

    h0, h1 = lax.fori_loop(0, k, hop, (h0, h1))
    return jnp.concatenate([h0[:N_NODES], h1[:N_NODES]], axis=1)


# speedup vs baseline: 5.5230x; 1.1015x over previous
import functools

import jax
import jax.numpy as jnp
from jax import lax
from jax.experimental import pallas as pl
from jax.experimental.pallas import tpu as pltpu
from jax.experimental.pallas import tpu_sc as plsc

N_NODES = 10000
N_PAD = 10240
D_FEAT = 256
N_EDGES = 160000
HALF = 128
NC = 2
NS = 16
ROWS_PER_TILE = N_PAD // NS

HOP_CH = 112
HOP_EPT = 10080
HOP_NCH = HOP_EPT // HOP_CH
HOP_NT = HOP_NCH // 2
DEG_CH = 64
DEG_EPT = 5120
DEG_NCH = DEG_EPT // DEG_CH
STG = 64
HSTG = 80


def _make_deg_kernel():
    mesh = plsc.VectorSubcoreMesh(core_axis_name="c", subcore_axis_name="s")

    @functools.partial(
        pl.kernel,
        mesh=mesh,
        out_type=jax.ShapeDtypeStruct((NC, N_PAD, HALF), jnp.float32),
        scratch_types=[
            pltpu.VMEM((DEG_NCH, DEG_CH), jnp.int32),
            pltpu.VMEM((DEG_CH, HALF), jnp.float32),
            pltpu.VMEM((STG, HALF), jnp.float32),
            pltpu.VMEM_SHARED((N_PAD, HALF), jnp.float32),
        ],
    )
    def deg_k(dst_hbm, ones_hbm, zeros_hbm, out_hbm,
              dst_v, ones_v, stage_v, acc_sh):
        c = lax.axis_index("c")
        s = lax.axis_index("s")
        wid = c * NS + s
        base = s * ROWS_PER_TILE

        pltpu.sync_copy(ones_hbm, ones_v)
        pltpu.sync_copy(zeros_hbm, stage_v)

        def zs(i, _):
            pltpu.sync_copy(stage_v, acc_sh.at[pl.ds(base + i * STG, STG)])
            return 0
        lax.fori_loop(0, ROWS_PER_TILE // STG, zs, 0)
        pltpu.sync_copy(dst_hbm.at[wid], dst_v)
        plsc.subcore_barrier()

        def step(j, _):
            pltpu.sync_copy(ones_v, acc_sh.at[dst_v.at[j]], add=True)
            return 0
        lax.fori_loop(0, DEG_NCH, step, 0)
        plsc.subcore_barrier()

        def outp(i, _):
            pltpu.sync_copy(acc_sh.at[pl.ds(base + i * STG, STG)], stage_v)
            pltpu.sync_copy(stage_v, out_hbm.at[c, pl.ds(base + i * STG, STG)])
            return 0
        lax.fori_loop(0, ROWS_PER_TILE // STG, outp, 0)

    return deg_k


def _make_hop_kernel():
    mesh = plsc.VectorSubcoreMesh(core_axis_name="c", subcore_axis_name="s")

    @functools.partial(
        pl.kernel,
        mesh=mesh,
        out_type=[
            jax.ShapeDtypeStruct((N_PAD, HALF), jnp.float32),
            jax.ShapeDtypeStruct((N_PAD, HALF), jnp.float32),
        ],
        scratch_types=[
            pltpu.VMEM((HOP_CH,), jnp.int32),
            pltpu.VMEM((HOP_CH,), jnp.int32),
            pltpu.VMEM((HOP_CH,), jnp.int32),
            pltpu.VMEM((HOP_CH,), jnp.int32),
            pltpu.VMEM((HOP_CH, HALF), jnp.float32),
            pltpu.VMEM((HOP_CH, HALF), jnp.float32),
            pltpu.VMEM_SHARED((N_PAD, HALF), jnp.float32),
            pltpu.SemaphoreType.DMA,
            pltpu.SemaphoreType.DMA,
            pltpu.SemaphoreType.DMA,
            pltpu.SemaphoreType.DMA,
            pltpu.SemaphoreType.DMA,
            pltpu.SemaphoreType.DMA,
        ],
    )
    def hop_k(h0_hbm, h1_hbm, src_hbm, dst_hbm, zeros_hbm, o0_hbm, o1_hbm,
              isa, isb, ida, idb, rows_a, rows_b, acc_sh, sem_a, sem_b,
              sem_isa, sem_isb, sem_ida, sem_idb):
        c = lax.axis_index("c")
        s = lax.axis_index("s")
        base = s * ROWS_PER_TILE

        pltpu.sync_copy(zeros_hbm, rows_a.at[pl.ds(0, HSTG)])

        def zs(i, _):
            pltpu.sync_copy(rows_a.at[pl.ds(0, HSTG)],
                            acc_sh.at[pl.ds(base + i * HSTG, HSTG)])
            return 0
        lax.fori_loop(0, ROWS_PER_TILE // HSTG, zs, 0)
        plsc.subcore_barrier()

        def run(h_hbm):
            def wait_a():
                pltpu.make_async_copy(
                    h_hbm.at[pl.ds(0, HOP_CH)], rows_a, sem_a).wait()

            def wait_b():
                pltpu.make_async_copy(
                    h_hbm.at[pl.ds(0, HOP_CH)], rows_b, sem_b).wait()

            def wait_idx(buf, sem):
                pltpu.make_async_copy(src_hbm.at[s, 0], buf, sem).wait()

            pltpu.async_copy(src_hbm.at[s, 0], isa, sem_isa)
            pltpu.async_copy(dst_hbm.at[s, 0], ida, sem_ida)
            pltpu.async_copy(src_hbm.at[s, 1], isb, sem_isb)
            pltpu.async_copy(dst_hbm.at[s, 1], idb, sem_idb)
            wait_idx(isa, sem_isa)
            pltpu.async_copy(h_hbm.at[isa], rows_a, sem_a)

            def step2(t, _):
                ja = 2 * t
                jb = ja + 1
                wait_a()

                @pl.when(t + 1 < HOP_NT)
                def _():
                    pltpu.async_copy(src_hbm.at[s, ja + 2], isa, sem_isa)

                wait_idx(isb, sem_isb)
                pltpu.async_copy(h_hbm.at[isb], rows_b, sem_b)
                wait_idx(ida, sem_ida)
                pltpu.sync_copy(rows_a, acc_sh.at[ida], add=True)

                @pl.when(t + 1 < HOP_NT)
                def _():
                    pltpu.async_copy(dst_hbm.at[s, ja + 2], ida, sem_ida)

                wait_b()

                @pl.when(t + 1 < HOP_NT)
                def _():
                    wait_idx(isa, sem_isa)
                    pltpu.async_copy(h_hbm.at[isa], rows_a, sem_a)
                    pltpu.async_copy(src_hbm.at[s, jb + 2], isb, sem_isb)

                wait_idx(idb, sem_idb)
                pltpu.sync_copy(rows_b, acc_sh.at[idb], add=True)

                @pl.when(t + 1 < HOP_NT)
                def _():
                    pltpu.async_copy(dst_hbm.at[s, jb + 2], idb, sem_idb)

                return 0
            lax.fori_loop(0, HOP_NT, step2, 0)

        @pl.when(c == 0)
        def _():
            run(h0_hbm)

        @pl.when(c == 1)
        def _():
            run(h1_hbm)

        plsc.subcore_barrier()

        def outp(o_hbm):
            def ostep(i, _):
                pltpu.sync_copy(acc_sh.at[pl.ds(base + i * HSTG, HSTG)],
                                rows_a.at[pl.ds(0, HSTG)])
                pltpu.sync_copy(rows_a.at[pl.ds(0, HSTG)],
                                o_hbm.at[pl.ds(base + i * HSTG, HSTG)])
                return 0
            lax.fori_loop(0, ROWS_PER_TILE // HSTG, ostep, 0)

        @pl.when(c == 0)
        def _():
            outp(o0_hbm)

        @pl.when(c == 1)
        def _():
            outp(o1_hbm)

    return hop_k


def _tc_body(s0_ref, s1_ref, inv_ref, w_ref, b_ref, a_ref, o0_ref, o1_ref):
    agg = jnp.concatenate([s0_ref[...], s1_ref[...]], axis=1) * inv_ref[...]
    lin = lax.dot_general(agg, w_ref[...], (((1,), (1,)), ((), ())),
                          preferred_element_type=jnp.float32)
    lin = lin + b_ref[...]
    a = a_ref[0, 0]
    out = jnp.maximum(lin, 0.0) + a * jnp.minimum(lin, 0.0)
    o0_ref[...] = out[:, :HALF]
    o1_ref[...] = out[:, HALF:]


def _tc_post(s0, s1, inv, w, b2, a2):
    r = 1024
    return pl.pallas_call(
        _tc_body,
        grid=(N_PAD // r,),
        in_specs=[
            pl.BlockSpec((r, HALF), lambda i: (i, 0)),
            pl.BlockSpec((r, HALF), lambda i: (i, 0)),
            pl.BlockSpec((r, 1), lambda i: (i, 0)),
            pl.BlockSpec((D_FEAT, D_FEAT), lambda i: (0, 0)),
            pl.BlockSpec((1, D_FEAT), lambda i: (0, 0)),
            pl.BlockSpec((1, 1), lambda i: (0, 0)),
        ],
        out_specs=[
            pl.BlockSpec((r, HALF), lambda i: (i, 0)),
            pl.BlockSpec((r, HALF), lambda i: (i, 0)),
        ],
        out_shape=[jax.ShapeDtypeStruct((N_PAD, HALF), jnp.float32)] * 2,
    )(s0, s1, inv, w, b2, a2)


_deg_kernel = _make_deg_kernel()
_hop_kernel = _make_hop_kernel()


def _pad_edges(idx, per_tile, padded_per_tile, n_tiles, nch, ch):
    t = idx.reshape(n_tiles, per_tile)
    t = jnp.pad(t, ((0, 0), (0, padded_per_tile - per_tile)),
                constant_values=N_NODES)
    return t.reshape(n_tiles, nch, ch)


def kernel(x, edge_index, k, W_weight, W_bias, prelu_a):
    ei = edge_index.astype(jnp.int32)
    src = _pad_edges(ei[0], N_EDGES // NS, HOP_EPT, NS, HOP_NCH, HOP_CH)
    dst = _pad_edges(ei[1], N_EDGES // NS, HOP_EPT, NS, HOP_NCH, HOP_CH)
    dst2 = _pad_edges(ei[1], N_EDGES // (NC * NS), DEG_EPT, NC * NS,
                      DEG_NCH, DEG_CH)
    h0 = jnp.pad(x[:, :HALF], ((0, N_PAD - N_NODES), (0, 0)))
    h1 = jnp.pad(x[:, HALF:], ((0, N_PAD - N_NODES), (0, 0)))

    ones_c = jnp.ones((DEG_CH, HALF), jnp.float32)
    zeros_c = jnp.zeros((STG, HALF), jnp.float32)
    zeros_h = jnp.zeros((HSTG, HALF), jnp.float32)
    degp = _deg_kernel(dst2, ones_c, zeros_c)
    deg = jnp.maximum(degp[0, :, 0] + degp[1, :, 0], 1.0)
    inv = (1.0 / deg).reshape(N_PAD, 1)
    b2 = W_bias.reshape(1, D_FEAT)
    a2 = prelu_a.reshape(1, 1)

    def hop(_, carry):
        c0, c1 = carry
        s0, s1 = _hop_kernel(c0, c1, src, dst, zeros_h)
        return tuple(_tc_post(s0, s1, inv, W_weight, b2, a2))

    h0, h1 = lax.fori_loop(0, k, hop, (h0, h1))
    return jnp.concatenate([h0[:N_NODES], h1[:N_NODES]], axis=1)
